# baseline (speedup 1.0000x reference)
import functools

import numpy as np
import jax
import jax.numpy as jnp
from jax.experimental import pallas as pl
from jax.experimental.pallas import tpu as pltpu

_LAT = 64
_NE = 1024
_HP = jax.lax.Precision.HIGHEST


def _mm(a, b):
    return jnp.dot(a.astype(jnp.bfloat16), b.astype(jnp.bfloat16),
                   preferred_element_type=jnp.float32)


def _mm_exact(a, b):
    return jnp.dot(a, b, precision=_HP, preferred_element_type=jnp.float32)


def _b16(a):
    return a.astype(jnp.bfloat16).astype(jnp.float32)


def _pad1_hw(x3):
    h, w, c = x3.shape
    zr = jnp.zeros((1, w, c), jnp.float32)
    x3 = jnp.concatenate([zr, x3, zr], axis=0)
    zc = jnp.zeros((h + 2, 1, c), jnp.float32)
    return jnp.concatenate([zc, x3, zc], axis=1)


def _stage1_body(xq_ref, w1q_ref, b1_ref, w2_ref, b2_ref, w3_ref, b3_ref,
                 cbt_ref, cb_ref, wd1_ref, bd1_ref, wd2_ref, bd2_ref,
                 hq_ref, yp_ref):
    xq = xq_ref[0]
    w1q = w1q_ref[...]
    b1 = b1_ref[...]
    w2 = w2_ref[...]
    b2 = b2_ref[...]
    w3 = w3_ref[...]
    b3 = b3_ref[...]
    cbt = cbt_ref[...]
    cb = cb_ref[...]

    planes = []
    for ph in range(2):
        row_pl = []
        for pw in range(2):
            acc = jnp.zeros((4096, 32), jnp.float32)
            for dv in range(2):
                for dw in range(2):
                    xs = xq[dv:dv + 64, dw:dw + 64, :].reshape(4096, 48)
                    acc = acc + _mm(xs, w1q[ph, pw, dv, dw])
            p = _b16(jnp.maximum(acc + b1, 0.0)).reshape(64, 64, 32)
            ia = jax.lax.broadcasted_iota(jnp.int32, (64, 64, 32), 0)
            ib = jax.lax.broadcasted_iota(jnp.int32, (64, 64, 32), 1)
            lo_r, hi_r = (1, 56) if ph == 0 else (0, 55)
            lo_c, hi_c = (1, 56) if pw == 0 else (0, 55)
            msk = (ia >= lo_r) & (ia <= hi_r) & (ib >= lo_c) & (ib <= hi_c)
            row_pl.append(jnp.where(msk, p, 0.0))
        planes.append(row_pl)

    acc = jnp.zeros((3136, 64), jnp.float32)
    for ph in range(2):
        for pw in range(2):
            for dh in range(2):
                for dw in range(2):
                    xs = planes[ph][pw][dh:dh + 56, dw:dw + 56, :]
                    acc = acc + _mm(xs.reshape(3136, 32), w2[2 * dh + ph, 2 * dw + pw])
    o2 = _b16(jnp.maximum(acc + b2, 0.0))

    o2p = _pad1_hw(o2.reshape(56, 56, 64))
    acc = jnp.zeros((3136, 64), jnp.float32)
    for u in range(3):
        for v in range(3):
            acc = acc + _mm(o2p[u:u + 56, v:v + 56, :].reshape(3136, 64), w3[u, v])
    h = acc + b3

    csum = jnp.sum(cbt * cbt, axis=0, keepdims=True)
    hq_parts = []
    for c0 in range(0, 3136, 784):
        hc = h[c0:c0 + 784]
        dist = (jnp.sum(hc * hc, axis=1, keepdims=True)
                - 2.0 * _mm(hc, cbt) + csum)
        m = jnp.min(dist, axis=1, keepdims=True)
        iota = jax.lax.broadcasted_iota(jnp.int32, (784, 1024), 1)
        idx = jnp.min(jnp.where(dist == m, iota, 2 ** 30), axis=1)
        oh = (iota == idx[:, None]).astype(jnp.float32)
        hq_parts.append(_mm_exact(oh, cb))
    hq = jnp.concatenate(hq_parts, axis=0)
    hq_ref[0] = hq

    hp = _pad1_hw(hq.reshape(56, 56, 64))
    acc = jnp.zeros((3136, 64), jnp.float32)
    for u in range(3):
        for v in range(3):
            acc = acc + _mm(hp[u:u + 56, v:v + 56, :].reshape(3136, 64), wd1_ref[u, v])
    d1 = jnp.maximum(acc + bd1_ref[...], 0.0)

    d1p = _pad1_hw(d1.reshape(56, 56, 64))
    for pi in range(2):
        for pj in range(2):
            acc = jnp.zeros((3136, 32), jnp.float32)
            for dh in range(2):
                for dw in range(2):
                    xs = d1p[pi + dh:pi + dh + 56, pj + dw:pj + dw + 56, :]
                    acc = acc + _mm(xs.reshape(3136, 64), wd2_ref[2 * dh + pi, 2 * dw + pj])
            yp_ref[0, 2 * pi + pj] = jnp.maximum(acc + bd2_ref[...], 0.0)


def _stage2_body(dp_ref, w3m_ref, b3a_ref, out_ref):
    for r in range(0, 112, 16):
        acc = jnp.zeros((16 * 112, 12), jnp.float32)
        for u in range(3):
            for v in range(3):
                xs = dp_ref[0, r + u:r + u + 16, v:v + 112, :]
                acc = acc + _mm(xs.reshape(16 * 112, 32), w3m_ref[u, v])
        out_ref[0, r * 112:(r + 16) * 112, :] = acc + b3a_ref[...]


def _full_spec(shape):
    rank = len(shape)
    return pl.BlockSpec(shape, lambda b, _r=rank: (0,) * _r)


@jax.jit
def kernel(x, W1, b1, W2, b2, W3, b3, codebook, Wd1, bd1, Wd2, bd2, Wd3, bd3):
    B = x.shape[0]

    xppp = jnp.pad(x, ((0, 0), (3, 61), (3, 61), (0, 0)))
    x1q = xppp.reshape(B, 72, 4, 72, 4, 3).transpose(0, 1, 3, 2, 4, 5)
    x1q = x1q.reshape(B, 72, 72, 48)

    S = np.zeros((2, 2, 4, 4), np.float32)
    for a in range(2):
        for dv in range(2):
            for kh in range(4):
                rh = kh + 2 * a - 4 * dv
                if 0 <= rh < 4:
                    S[a, dv, kh, rh] = 1.0
    S = jnp.asarray(S)
    w1q = jnp.einsum('advr,bews,vwco->abdersco', S, S, W1,
                     precision=_HP).reshape(2, 2, 2, 2, 48, 32)

    T = np.zeros((3, 2, 4), np.float32)
    for u in range(3):
        for p in range(2):
            dh = u - p
            if 0 <= dh < 2:
                T[u, p, 2 * dh + p] = 1.0
    T = jnp.asarray(T)
    w3m = jnp.einsum('upk,vql,klic->uvipqc', T, T, Wd3,
                     precision=_HP).reshape(3, 3, 32, 12)
    b3a = jnp.tile(bd3, 4)

    cbt = codebook.T

    grid = (B,)
    hq, yp = pl.pallas_call(
        _stage1_body,
        grid=grid,
        in_specs=[
            pl.BlockSpec((1, 72, 72, 48), lambda b: (b, 0, 0, 0)),
            _full_spec((2, 2, 2, 2, 48, 32)),
            _full_spec((32,)),
            _full_spec((4, 4, 32, 64)),
            _full_spec((64,)),
            _full_spec((3, 3, 64, 64)),
            _full_spec((64,)),
            _full_spec((64, 1024)),
            _full_spec((1024, 64)),
            _full_spec((3, 3, 64, 64)),
            _full_spec((64,)),
            _full_spec((4, 4, 64, 32)),
            _full_spec((32,)),
        ],
        out_specs=[
            pl.BlockSpec((1, 3136, 64), lambda b: (b, 0, 0)),
            pl.BlockSpec((1, 4, 3136, 32), lambda b: (b, 0, 0, 0)),
        ],
        out_shape=[
            jax.ShapeDtypeStruct((B, 3136, 64), jnp.float32),
            jax.ShapeDtypeStruct((B, 4, 3136, 32), jnp.float32),
        ],
        compiler_params=pltpu.CompilerParams(
            vmem_limit_bytes=110 * 1024 * 1024,
        ),
    )(x1q, w1q, b1, W2, b2, W3, b3, cbt, codebook, Wd1, bd1, Wd2, bd2)

    d2 = yp.reshape(B, 2, 2, 56, 56, 32).transpose(0, 3, 1, 4, 2, 5)
    d2 = d2.reshape(B, 112, 112, 32)
    d2p = jnp.pad(d2, ((0, 0), (1, 1), (1, 1), (0, 0)))

    outall = pl.pallas_call(
        _stage2_body,
        grid=grid,
        in_specs=[
            pl.BlockSpec((1, 114, 114, 32), lambda b: (b, 0, 0, 0)),
            _full_spec((3, 3, 32, 12)),
            _full_spec((12,)),
        ],
        out_specs=pl.BlockSpec((1, 12544, 12), lambda b: (b, 0, 0)),
        out_shape=jax.ShapeDtypeStruct((B, 12544, 12), jnp.float32),
        compiler_params=pltpu.CompilerParams(
            vmem_limit_bytes=110 * 1024 * 1024,
        ),
    )(d2p, w3m, b3a)

    d = outall.reshape(B, 112, 112, 2, 2, 3).transpose(0, 1, 3, 2, 4, 5)
    d = d.reshape(B, 224, 224, 3)
    h_q = hq.reshape(B, 56, 56, _LAT)
    return (h_q, d)


# speedup vs baseline: 1.0994x; 1.0994x over previous
import functools

import numpy as np
import jax
import jax.numpy as jnp
from jax.experimental import pallas as pl
from jax.experimental.pallas import tpu as pltpu

_LAT = 64
_NE = 1024
_HP = jax.lax.Precision.HIGHEST


def _mm(a, b):
    return jnp.dot(a.astype(jnp.bfloat16), b.astype(jnp.bfloat16),
                   preferred_element_type=jnp.float32)


def _mm_exact(a, b):
    return jnp.dot(a, b, precision=_HP, preferred_element_type=jnp.float32)


def _b16(a):
    return a.astype(jnp.bfloat16).astype(jnp.float32)


def _pad1_hw(x3):
    h, w, c = x3.shape
    zr = jnp.zeros((1, w, c), jnp.float32)
    x3 = jnp.concatenate([zr, x3, zr], axis=0)
    zc = jnp.zeros((h + 2, 1, c), jnp.float32)
    return jnp.concatenate([zc, x3, zc], axis=1)


def _stage1_body(xq_ref, w1q_ref, b1_ref, w2_ref, b2_ref, w3_ref, b3_ref,
                 cbt_ref, cb_ref, wd1_ref, bd1_ref, wd2_ref, bd2_ref,
                 hq_ref, yp_ref):
    xq = xq_ref[0]
    w1q = w1q_ref[...]
    b1 = b1_ref[...]
    w2 = w2_ref[...]
    b2 = b2_ref[...]
    w3 = w3_ref[...]
    b3 = b3_ref[...]
    cbt = cbt_ref[...]
    cb = cb_ref[...]

    planes = []
    for ph in range(2):
        row_pl = []
        for pw in range(2):
            acc = jnp.zeros((4096, 32), jnp.float32)
            for dv in range(2):
                for dw in range(2):
                    xs = xq[dv:dv + 64, dw:dw + 64, :].reshape(4096, 48)
                    acc = acc + _mm(xs, w1q[ph, pw, dv, dw])
            p = _b16(jnp.maximum(acc + b1, 0.0)).reshape(64, 64, 32)
            ia = jax.lax.broadcasted_iota(jnp.int32, (64, 64, 32), 0)
            ib = jax.lax.broadcasted_iota(jnp.int32, (64, 64, 32), 1)
            lo_r, hi_r = (1, 56) if ph == 0 else (0, 55)
            lo_c, hi_c = (1, 56) if pw == 0 else (0, 55)
            msk = (ia >= lo_r) & (ia <= hi_r) & (ib >= lo_c) & (ib <= hi_c)
            row_pl.append(jnp.where(msk, p, 0.0))
        planes.append(row_pl)

    acc = jnp.zeros((3136, 64), jnp.float32)
    for ph in range(2):
        for pw in range(2):
            for dh in range(2):
                for dw in range(2):
                    xs = planes[ph][pw][dh:dh + 56, dw:dw + 56, :]
                    acc = acc + _mm(xs.reshape(3136, 32), w2[2 * dh + ph, 2 * dw + pw])
    o2 = _b16(jnp.maximum(acc + b2, 0.0))

    o2p = _pad1_hw(o2.reshape(56, 56, 64))
    acc = jnp.zeros((3136, 64), jnp.float32)
    for u in range(3):
        for v in range(3):
            acc = acc + _mm(o2p[u:u + 56, v:v + 56, :].reshape(3136, 64), w3[u, v])
    h = acc + b3

    csum = jnp.sum(cbt * cbt, axis=0, keepdims=True)
    hq_parts = []
    for c0 in range(0, 3136, 784):
        hc = h[c0:c0 + 784]
        dist = (jnp.sum(hc * hc, axis=1, keepdims=True)
                - 2.0 * _mm(hc, cbt) + csum)
        m = jnp.min(dist, axis=1, keepdims=True)
        iota = jax.lax.broadcasted_iota(jnp.int32, (784, 1024), 1)
        idx = jnp.min(jnp.where(dist == m, iota, 2 ** 30), axis=1)
        oh = (iota == idx[:, None]).astype(jnp.float32)
        hq_parts.append(_mm(oh, cb))
    hq = jnp.concatenate(hq_parts, axis=0)
    hq_ref[0] = hq

    hp = _pad1_hw(hq.reshape(56, 56, 64))
    acc = jnp.zeros((3136, 64), jnp.float32)
    for u in range(3):
        for v in range(3):
            acc = acc + _mm(hp[u:u + 56, v:v + 56, :].reshape(3136, 64), wd1_ref[u, v])
    d1 = jnp.maximum(acc + bd1_ref[...], 0.0)

    d1p = _pad1_hw(d1.reshape(56, 56, 64))
    for pi in range(2):
        for pj in range(2):
            acc = jnp.zeros((3136, 32), jnp.float32)
            for dh in range(2):
                for dw in range(2):
                    xs = d1p[pi + dh:pi + dh + 56, pj + dw:pj + dw + 56, :]
                    acc = acc + _mm(xs.reshape(3136, 64), wd2_ref[2 * dh + pi, 2 * dw + pj])
            yp_ref[0, 2 * pi + pj] = jnp.maximum(acc + bd2_ref[...], 0.0)


def _stage2_body(dp_ref, w3m_ref, b3a_ref, out_ref):
    for r in range(0, 112, 16):
        acc = jnp.zeros((16 * 112, 12), jnp.float32)
        for u in range(3):
            for v in range(3):
                xs = dp_ref[0, r + u:r + u + 16, v:v + 112, :]
                acc = acc + _mm(xs.reshape(16 * 112, 32), w3m_ref[u, v])
        out_ref[0, r * 112:(r + 16) * 112, :] = acc + b3a_ref[...]


def _full_spec(shape):
    rank = len(shape)
    return pl.BlockSpec(shape, lambda b, _r=rank: (0,) * _r)


@jax.jit
def kernel(x, W1, b1, W2, b2, W3, b3, codebook, Wd1, bd1, Wd2, bd2, Wd3, bd3):
    B = x.shape[0]

    xppp = jnp.pad(x, ((0, 0), (3, 61), (3, 61), (0, 0)))
    x1q = xppp.reshape(B, 72, 4, 72, 4, 3).transpose(0, 1, 3, 2, 4, 5)
    x1q = x1q.reshape(B, 72, 72, 48)

    S = np.zeros((2, 2, 4, 4), np.float32)
    for a in range(2):
        for dv in range(2):
            for kh in range(4):
                rh = kh + 2 * a - 4 * dv
                if 0 <= rh < 4:
                    S[a, dv, kh, rh] = 1.0
    S = jnp.asarray(S)
    w1q = jnp.einsum('advr,bews,vwco->abdersco', S, S, W1,
                     precision=_HP).reshape(2, 2, 2, 2, 48, 32)

    T = np.zeros((3, 2, 4), np.float32)
    for u in range(3):
        for p in range(2):
            dh = u - p
            if 0 <= dh < 2:
                T[u, p, 2 * dh + p] = 1.0
    T = jnp.asarray(T)
    w3m = jnp.einsum('upk,vql,klic->uvipqc', T, T, Wd3,
                     precision=_HP).reshape(3, 3, 32, 12)
    b3a = jnp.tile(bd3, 4)

    cbt = codebook.T

    grid = (B,)
    hq, yp = pl.pallas_call(
        _stage1_body,
        grid=grid,
        in_specs=[
            pl.BlockSpec((1, 72, 72, 48), lambda b: (b, 0, 0, 0)),
            _full_spec((2, 2, 2, 2, 48, 32)),
            _full_spec((32,)),
            _full_spec((4, 4, 32, 64)),
            _full_spec((64,)),
            _full_spec((3, 3, 64, 64)),
            _full_spec((64,)),
            _full_spec((64, 1024)),
            _full_spec((1024, 64)),
            _full_spec((3, 3, 64, 64)),
            _full_spec((64,)),
            _full_spec((4, 4, 64, 32)),
            _full_spec((32,)),
        ],
        out_specs=[
            pl.BlockSpec((1, 3136, 64), lambda b: (b, 0, 0)),
            pl.BlockSpec((1, 4, 3136, 32), lambda b: (b, 0, 0, 0)),
        ],
        out_shape=[
            jax.ShapeDtypeStruct((B, 3136, 64), jnp.float32),
            jax.ShapeDtypeStruct((B, 4, 3136, 32), jnp.float32),
        ],
        compiler_params=pltpu.CompilerParams(
            vmem_limit_bytes=110 * 1024 * 1024,
            dimension_semantics=("parallel",),
        ),
    )(x1q, w1q, b1, W2, b2, W3, b3, cbt, codebook, Wd1, bd1, Wd2, bd2)

    d2 = yp.reshape(B, 2, 2, 56, 56, 32).transpose(0, 3, 1, 4, 2, 5)
    d2 = d2.reshape(B, 112, 112, 32)
    d2p = jnp.pad(d2, ((0, 0), (1, 1), (1, 1), (0, 0)))

    outall = pl.pallas_call(
        _stage2_body,
        grid=grid,
        in_specs=[
            pl.BlockSpec((1, 114, 114, 32), lambda b: (b, 0, 0, 0)),
            _full_spec((3, 3, 32, 12)),
            _full_spec((12,)),
        ],
        out_specs=pl.BlockSpec((1, 12544, 12), lambda b: (b, 0, 0)),
        out_shape=jax.ShapeDtypeStruct((B, 12544, 12), jnp.float32),
        compiler_params=pltpu.CompilerParams(
            vmem_limit_bytes=110 * 1024 * 1024,
            dimension_semantics=("parallel",),
        ),
    )(d2p, w3m, b3a)

    d = outall.reshape(B, 112, 112, 2, 2, 3).transpose(0, 1, 3, 2, 4, 5)
    d = d.reshape(B, 224, 224, 3)
    h_q = hq.reshape(B, 56, 56, _LAT)
    return (h_q, d)
